# baseline (speedup 1.0000x reference)
import jax
import jax.numpy as jnp
from jax.experimental import pallas as pl
from jax.experimental.pallas import tpu as pltpu

_IN = 32
_OUT = 32
_TILE = 512
_BSPLIT = 2


def _graphconv_body(a_ref, f_ref, w_ref, b_ref, o_ref):
    i = pl.program_id(1)
    a = a_ref[0]
    f = f_ref[0]
    agg = jnp.dot(a, f, preferred_element_type=jnp.float32)
    ft = f_ref[0, pl.ds(i * _TILE, _TILE), :]
    w1 = w_ref[:_IN, :]
    w2 = w_ref[_IN:, :]
    out = (jnp.dot(ft, w1, preferred_element_type=jnp.float32)
           + jnp.dot(agg, w2, preferred_element_type=jnp.float32)
           + b_ref[...])
    o_ref[0] = jnp.maximum(out, 0.0)


def kernel(features, A, weight, bias):
    B, N, I, IN = features.shape
    OUT = weight.shape[1]
    bias2d = bias.reshape(1, OUT)
    bh = B // _BSPLIT

    halves = []
    for h in range(_BSPLIT):
        f2d_h = features[h * bh:(h + 1) * bh].reshape(bh, N * I, IN)
        grid = (bh, N // _TILE)
        out_h = pl.pallas_call(
            _graphconv_body,
            grid=grid,
            in_specs=[
                pl.BlockSpec((1, _TILE, N),
                             lambda b, i, h=h: (b + h * bh, i, 0)),
                pl.BlockSpec((1, N, IN), lambda b, i: (b, 0, 0)),
                pl.BlockSpec((weight.shape[0], OUT), lambda b, i: (0, 0)),
                pl.BlockSpec((1, OUT), lambda b, i: (0, 0)),
            ],
            out_specs=pl.BlockSpec((1, _TILE, OUT), lambda b, i: (b, i, 0)),
            out_shape=jax.ShapeDtypeStruct((bh, N, OUT), jnp.float32),
            compiler_params=pltpu.CompilerParams(
                dimension_semantics=("parallel", "arbitrary")),
        )(A, f2d_h, weight, bias2d)
        halves.append(out_h.reshape(bh, N, I, OUT))
    return jnp.concatenate(halves, axis=0)


# speedup vs baseline: 1.0522x; 1.0522x over previous
import jax
import jax.numpy as jnp
from jax.experimental import pallas as pl
from jax.experimental.pallas import tpu as pltpu

_IN = 32
_OUT = 32
_TILE = 512


def _graphconv_body(a_ref, f_ref, w_ref, b_ref, o_ref):
    i = pl.program_id(1)
    a = a_ref[0]
    f = f_ref[0]
    agg = jnp.dot(a, f, preferred_element_type=jnp.float32)
    ft = f_ref[0, pl.ds(i * _TILE, _TILE), :]
    w1 = w_ref[:_IN, :]
    w2 = w_ref[_IN:, :]
    out = (jnp.dot(ft, w1, preferred_element_type=jnp.float32)
           + jnp.dot(agg, w2, preferred_element_type=jnp.float32)
           + b_ref[...])
    o_ref[0] = jnp.maximum(out, 0.0)


def kernel(features, A, weight, bias):
    B, N, I, IN = features.shape
    OUT = weight.shape[1]
    f2d = features.reshape(B, N * I, IN)
    bias2d = bias.reshape(1, OUT)

    grid = (B, N // _TILE)
    out = pl.pallas_call(
        _graphconv_body,
        grid=grid,
        in_specs=[
            pl.BlockSpec((1, _TILE, N), lambda b, i: (b, i, 0)),
            pl.BlockSpec((1, N, IN), lambda b, i: (b, 0, 0)),
            pl.BlockSpec((weight.shape[0], OUT), lambda b, i: (0, 0)),
            pl.BlockSpec((1, OUT), lambda b, i: (0, 0)),
        ],
        out_specs=pl.BlockSpec((1, _TILE, OUT), lambda b, i: (b, i, 0)),
        out_shape=jax.ShapeDtypeStruct((B, N, OUT), jnp.float32),
        compiler_params=pltpu.CompilerParams(
            dimension_semantics=("parallel", "arbitrary")),
    )(A, f2d, weight, bias2d)
    return out.reshape(B, N, I, OUT)
